# baseline (speedup 1.0000x reference)
import functools

import jax
import jax.numpy as jnp
from jax import lax
from jax.experimental import pallas as pl
from jax.experimental.pallas import tpu as pltpu
from jax.experimental.pallas import tpu_sc as plsc

_B = 16
_P = 1024
_D = 1024
_ROWS = _B * _P
_NW = 32
_RPW = _ROWS // _NW
_TILE = 64
_NCH = _RPW // _TILE

_mesh = plsc.VectorSubcoreMesh(core_axis_name="c", subcore_axis_name="s")


@functools.partial(
    pl.kernel,
    mesh=_mesh,
    out_type=jax.ShapeDtypeStruct((_ROWS, _D), jnp.float32),
    scratch_types=[
        pltpu.VMEM((_TILE,), jnp.int32),
        pltpu.VMEM((_TILE, _D), jnp.float32),
        pltpu.SemaphoreType.DMA,
        pltpu.SemaphoreType.DMA,
    ],
)
def _sc_broadcast(idx_hbm, table_hbm, out_hbm, idx_v, buf_v, gsem, osem):
    wid = lax.axis_index("s") * 2 + lax.axis_index("c")
    base = wid * _RPW
    pltpu.sync_copy(idx_hbm, idx_v)
    pltpu.async_copy(table_hbm.at[idx_v], buf_v, gsem).wait()
    copies = [
        pltpu.async_copy(
            buf_v, out_hbm.at[pl.ds(base + j * _TILE, _TILE)], osem
        )
        for j in range(_NCH)
    ]
    for c in copies:
        c.wait()


def kernel(scale_embed, batch_size, num_patches, scale_idx):
    dep = (jnp.asarray(batch_size) - _B) + (jnp.asarray(num_patches) - _P)
    idx = (jnp.asarray(scale_idx) + dep).astype(jnp.int32)
    idx_arr = jnp.broadcast_to(idx, (_TILE,))
    out2d = _sc_broadcast(idx_arr, scale_embed)
    return out2d.reshape(_B, _P, _D)


# speedup vs baseline: 1.7958x; 1.7958x over previous
import functools

import jax
import jax.numpy as jnp
from jax import lax
from jax.experimental import pallas as pl
from jax.experimental.pallas import tpu as pltpu
from jax.experimental.pallas import tpu_sc as plsc

_B = 16
_P = 1024
_D = 1024
_ROWS = _B * _P
_NW = 32
_RPW = _ROWS // _NW
_GTILE = 64
_NREP = _RPW // _GTILE

_mesh = plsc.VectorSubcoreMesh(core_axis_name="c", subcore_axis_name="s")


@functools.partial(
    pl.kernel,
    mesh=_mesh,
    out_type=jax.ShapeDtypeStruct((_ROWS, _D), jnp.float32),
    scratch_types=[
        pltpu.VMEM((_GTILE,), jnp.int32),
        pltpu.VMEM((_GTILE, _D), jnp.float32),
        pltpu.VMEM_SHARED((_RPW, _D), jnp.float32),
        pltpu.SemaphoreType.DMA,
        pltpu.SemaphoreType.DMA,
    ],
)
def _sc_broadcast(idx_hbm, table_hbm, out_hbm, idx_v, buf_v, stage_s, gsem, osem):
    cid = lax.axis_index("c")
    sid = lax.axis_index("s")
    wid = sid * 2 + cid
    base = wid * _RPW

    @pl.when(sid == 0)
    def _fill_stage():
        pltpu.sync_copy(idx_hbm, idx_v)
        pltpu.async_copy(table_hbm.at[idx_v], buf_v, gsem).wait()
        for j in range(_NREP):
            pltpu.sync_copy(buf_v, stage_s.at[pl.ds(j * _GTILE, _GTILE)])

    plsc.subcore_barrier()
    pltpu.async_copy(stage_s, out_hbm.at[pl.ds(base, _RPW)], osem).wait()


def kernel(scale_embed, batch_size, num_patches, scale_idx):
    dep = (jnp.asarray(batch_size) - _B) + (jnp.asarray(num_patches) - _P)
    idx = (jnp.asarray(scale_idx) + dep).astype(jnp.int32)
    idx_arr = jnp.broadcast_to(idx, (_GTILE,))
    out2d = _sc_broadcast(idx_arr, scale_embed)
    return out2d.reshape(_B, _P, _D)
